# Initial kernel scaffold; baseline (speedup 1.0000x reference)
import jax
import jax.numpy as jnp
from jax.experimental import pallas as pl


def kernel(input, edge_index, edge_weight, W):
    raise NotImplementedError("write your pallas kernel here")


# speedup vs baseline: 4.1044x; 4.1044x over previous
import functools

import jax
import jax.numpy as jnp
from jax import lax
from jax.experimental import pallas as pl
from jax.experimental.pallas import tpu as pltpu
from jax.experimental.pallas import tpu_sc as plsc

L = 16
NC = 2
NS = 16


def _matmul_body(x_ref, wt_ref, o_ref):
    o_ref[...] = jnp.dot(x_ref[...], wt_ref[...],
                         preferred_element_type=jnp.float32)


def _support_matmul(x, wt):
    n, d_in = x.shape
    d_out = wt.shape[1]
    blk = 1000
    return pl.pallas_call(
        _matmul_body,
        grid=(n // blk,),
        in_specs=[pl.BlockSpec((blk, d_in), lambda i: (i, 0)),
                  pl.BlockSpec((d_in, d_out), lambda i: (0, 0))],
        out_specs=pl.BlockSpec((blk, d_out), lambda i: (i, 0)),
        out_shape=jax.ShapeDtypeStruct((n, d_out), jnp.float32),
    )(x, wt)


def _combine_body(p_ref, o_ref):
    o_ref[...] = p_ref[0] + p_ref[1]


def _combine(partials, n):
    _, _, d = partials.shape
    blk = 1000
    return pl.pallas_call(
        _combine_body,
        grid=(n // blk,),
        in_specs=[pl.BlockSpec((2, blk, d), lambda i: (0, i, 0))],
        out_specs=pl.BlockSpec((blk, d), lambda i: (i, 0)),
        out_shape=jax.ShapeDtypeStruct((n, d), jnp.float32),
    )(partials)


@functools.lru_cache(maxsize=None)
def _make_sc_scatter(n, e, d):
    nw = NC * NS
    ept = e // nw
    assert e % nw == 0 and ept % 8 == 0
    k = 80
    assert ept % k == 0
    n_chunks = ept // k
    zr = 128
    rpt = -(-n // (NS * zr)) * zr
    n_pad = rpt * NS
    mesh = plsc.VectorSubcoreMesh(core_axis_name="c", subcore_axis_name="s",
                                  num_cores=NC, num_subcores=NS)

    @functools.partial(
        pl.kernel,
        out_type=jax.ShapeDtypeStruct((NC, n_pad, d), jnp.float32),
        mesh=mesh,
        scratch_types=[
            pltpu.VMEM_SHARED((n_pad, d), jnp.float32),
            pltpu.VMEM((k,), jnp.int32),
            pltpu.VMEM((k,), jnp.int32),
            pltpu.VMEM((k,), jnp.float32),
            pltpu.VMEM((k, d), jnp.float32),
            pltpu.VMEM((zr, d), jnp.float32),
        ],
    )
    def sc_kernel(support, src, dst, w, out, acc, src_v, dst_v, w_v, rows,
                  stage):
        c = lax.axis_index("c")
        s = lax.axis_index("s")
        wid = s * NC + c
        zero = jnp.zeros((L,), jnp.float32)

        def zrow(r, carry):
            for j in range(d // L):
                stage[r, pl.ds(j * L, L)] = zero
            return carry
        lax.fori_loop(0, zr, zrow, 0)

        def zcopy(b, carry):
            pltpu.sync_copy(stage, acc.at[pl.ds(s * rpt + b * zr, zr)])
            return carry
        lax.fori_loop(0, rpt // zr, zcopy, 0)
        plsc.subcore_barrier()

        base = wid * ept

        def chunk_body(ci, carry):
            off = base + ci * k
            pltpu.sync_copy(src.at[pl.ds(off, k)], src_v)
            pltpu.sync_copy(dst.at[pl.ds(off, k)], dst_v)
            pltpu.sync_copy(w.at[pl.ds(off, k)], w_v)
            pltpu.sync_copy(support.at[src_v], rows)

            dn = lax.GatherDimensionNumbers(
                offset_dims=(), collapsed_slice_dims=(0,),
                start_index_map=(0,))

            def group_body(g, cc):
                w_reg = w_v[pl.ds(g * L, L)]

                def lane_body(l, cc2):
                    i = g * L + l
                    widx = jnp.full((L,), l, jnp.int32)
                    wvec = lax.gather(
                        w_reg, widx[:, None], dn, slice_sizes=(1,),
                        mode=lax.GatherScatterMode.PROMISE_IN_BOUNDS)
                    for j in range(d // L):
                        sl = pl.ds(j * L, L)
                        rows[i, sl] = rows[i, sl] * wvec
                    return cc2
                lax.fori_loop(0, L, lane_body, 0)
                return cc
            lax.fori_loop(0, k // L, group_body, 0)
            pltpu.sync_copy(rows, acc.at[dst_v], add=True)
            return carry
        lax.fori_loop(0, n_chunks, chunk_body, 0)
        plsc.subcore_barrier()

        def wout(b, carry):
            r0 = s * rpt + b * zr
            pltpu.sync_copy(acc.at[pl.ds(r0, zr)], stage)
            pltpu.sync_copy(stage, out.at[c, pl.ds(r0, zr)])
            return carry
        lax.fori_loop(0, rpt // zr, wout, 0)

    return sc_kernel


def kernel(input, edge_index, edge_weight, W):
    n, _ = input.shape
    d_out = W.shape[0]
    e = edge_weight.shape[0]
    support = _support_matmul(input, W.T)
    dst = edge_index[0]
    src = edge_index[1]
    partials = _make_sc_scatter(n, e, d_out)(support, src, dst, edge_weight)
    return _combine(partials, n)


# speedup vs baseline: 5.6857x; 1.3853x over previous
import functools

import jax
import jax.numpy as jnp
from jax import lax
from jax.experimental import pallas as pl
from jax.experimental.pallas import tpu as pltpu
from jax.experimental.pallas import tpu_sc as plsc

L = 16
NC = 2
NS = 16


def _matmul_body(x_ref, wt_ref, o_ref):
    o_ref[...] = jnp.dot(x_ref[...], wt_ref[...],
                         preferred_element_type=jnp.float32)


def _support_matmul(x, wt):
    n, d_in = x.shape
    d_out = wt.shape[1]
    blk = 1000
    return pl.pallas_call(
        _matmul_body,
        grid=(n // blk,),
        in_specs=[pl.BlockSpec((blk, d_in), lambda i: (i, 0)),
                  pl.BlockSpec((d_in, d_out), lambda i: (0, 0))],
        out_specs=pl.BlockSpec((blk, d_out), lambda i: (i, 0)),
        out_shape=jax.ShapeDtypeStruct((n, d_out), jnp.float32),
    )(x, wt)


def _combine_body(p_ref, o_ref):
    o_ref[...] = p_ref[0] + p_ref[1]


def _combine(partials, n):
    _, _, d = partials.shape
    blk = 1000
    return pl.pallas_call(
        _combine_body,
        grid=(n // blk,),
        in_specs=[pl.BlockSpec((2, blk, d), lambda i: (0, i, 0))],
        out_specs=pl.BlockSpec((blk, d), lambda i: (i, 0)),
        out_shape=jax.ShapeDtypeStruct((n, d), jnp.float32),
    )(partials)


@functools.lru_cache(maxsize=None)
def _make_sc_scatter(n, e, d):
    nw = NC * NS
    ept = e // nw
    assert e % nw == 0 and ept % 8 == 0
    k = 80
    assert ept % k == 0
    n_chunks = ept // k
    zr = 128
    rpt = -(-n // (NS * zr)) * zr
    n_pad = rpt * NS
    mesh = plsc.VectorSubcoreMesh(core_axis_name="c", subcore_axis_name="s",
                                  num_cores=NC, num_subcores=NS)

    @functools.partial(
        pl.kernel,
        out_type=jax.ShapeDtypeStruct((NC, n_pad, d), jnp.float32),
        mesh=mesh,
        scratch_types=[
            pltpu.VMEM_SHARED((n_pad, d), jnp.float32),
            pltpu.VMEM((2, k), jnp.int32),
            pltpu.VMEM((2, k), jnp.int32),
            pltpu.VMEM((2, k), jnp.float32),
            pltpu.VMEM((2, k, d), jnp.float32),
            pltpu.VMEM((zr, d), jnp.float32),
            pltpu.SemaphoreType.DMA,
        ],
    )
    def sc_kernel(support, src, dst, w, out, acc, src_v, dst_v, w_v, rows,
                  stage, sem):
        c = lax.axis_index("c")
        s = lax.axis_index("s")
        wid = s * NC + c
        zero = jnp.zeros((L,), jnp.float32)

        def zrow(r, carry):
            for j in range(d // L):
                stage[r, pl.ds(j * L, L)] = zero
            return carry
        lax.fori_loop(0, zr, zrow, 0)

        def zcopy(b, carry):
            pltpu.sync_copy(stage, acc.at[pl.ds(s * rpt + b * zr, zr)])
            return carry
        lax.fori_loop(0, rpt // zr, zcopy, 0)
        plsc.subcore_barrier()

        base = wid * ept

        def load_idx(ci, buf):
            off = base + ci * k
            pltpu.sync_copy(src.at[pl.ds(off, k)], src_v.at[buf])
            pltpu.sync_copy(dst.at[pl.ds(off, k)], dst_v.at[buf])
            pltpu.sync_copy(w.at[pl.ds(off, k)], w_v.at[buf])

        def start_gather(buf):
            pltpu.async_copy(support.at[src_v.at[buf]], rows.at[buf], sem)

        def wait_gather(buf):
            pltpu.make_async_copy(support.at[src_v.at[buf]], rows.at[buf],
                                  sem).wait()

        load_idx(0, 0)
        start_gather(0)

        dn = lax.GatherDimensionNumbers(
            offset_dims=(), collapsed_slice_dims=(0,), start_index_map=(0,))

        def chunk_body(ci, carry):
            b = ci % 2

            @pl.when(ci + 1 < n_chunks)
            def _():
                load_idx(ci + 1, 1 - b)
                start_gather(1 - b)

            wait_gather(b)

            def group_body(g, cc):
                w_reg = w_v[b, pl.ds(g * L, L)]

                def lane_body(l, cc2):
                    i = g * L + l
                    widx = jnp.full((L,), l, jnp.int32)
                    wvec = lax.gather(
                        w_reg, widx[:, None], dn, slice_sizes=(1,),
                        mode=lax.GatherScatterMode.PROMISE_IN_BOUNDS)
                    for j in range(d // L):
                        sl = pl.ds(j * L, L)
                        rows[b, i, sl] = rows[b, i, sl] * wvec
                    return cc2
                lax.fori_loop(0, L, lane_body, 0)
                return cc
            lax.fori_loop(0, k // L, group_body, 0)
            pltpu.sync_copy(rows.at[b], acc.at[dst_v.at[b]], add=True)
            return carry
        lax.fori_loop(0, n_chunks, chunk_body, 0)
        plsc.subcore_barrier()

        def wout(bb, carry):
            r0 = s * rpt + bb * zr
            pltpu.sync_copy(acc.at[pl.ds(r0, zr)], stage)
            pltpu.sync_copy(stage, out.at[c, pl.ds(r0, zr)])
            return carry
        lax.fori_loop(0, rpt // zr, wout, 0)

    return sc_kernel


def kernel(input, edge_index, edge_weight, W):
    n, _ = input.shape
    d_out = W.shape[0]
    e = edge_weight.shape[0]
    support = _support_matmul(input, W.T)
    dst = edge_index[0]
    src = edge_index[1]
    partials = _make_sc_scatter(n, e, d_out)(support, src, dst, edge_weight)
    return _combine(partials, n)
